# baseline (speedup 1.0000x reference)
import jax
import jax.numpy as jnp
from jax.experimental import pallas as pl
from jax.experimental.pallas import tpu as pltpu

_DIM = 128
_NH = 8
_NBUCKETS = 1024.0
_BLK = 4000


def _lsh_block(x_ref, rv_ref, o_ref):
    proj = jnp.dot(x_ref[...], rv_ref[...], preferred_element_type=jnp.float32)
    o_ref[...] = jnp.floor(proj) % _NBUCKETS


def kernel(x, random_vectors):
    n = x.shape[0]
    grid = (n // _BLK,)
    return pl.pallas_call(
        _lsh_block,
        grid=grid,
        in_specs=[
            pl.BlockSpec((_BLK, _DIM), lambda i: (i, 0)),
            pl.BlockSpec((_DIM, _NH), lambda i: (0, 0)),
        ],
        out_specs=pl.BlockSpec((_BLK, _NH), lambda i: (i, 0)),
        out_shape=jax.ShapeDtypeStruct((n, _NH), jnp.float32),
        compiler_params=pltpu.CompilerParams(dimension_semantics=("parallel",)),
    )(x, random_vectors)
